# baseline (speedup 1.0000x reference)
import functools

import jax
import jax.numpy as jnp
import numpy as np
from jax.experimental import pallas as pl

IMG_SIZE = 1024.0
NPTS = 250
PCH = 256
PHALF = 128

_SIN_C = tuple(np.float32(v) for v in (
    6.283183466376198, -41.34148035624613, 81.59765787614148,
    -76.59492821657112, 41.269929567669145, -12.372494818439662))
_COS_C = tuple(np.float32(v) for v in (
    0.999999992290297, -19.73920555404448, 64.93917223259542,
    -85.45116579292082, 60.176230338868066, -26.000527873748382,
    6.575611642718274))


def _body(xy1_ref, lab_ref, w_ref, tab_ref, out_ref):
    br = xy1_ref.shape[0]
    u = jax.lax.dot_general(
        xy1_ref[...], w_ref[...],
        dimension_numbers=(((1,), (0,)), ((), ())),
        precision=jax.lax.Precision.HIGHEST,
        preferred_element_type=jnp.float32)
    f = u - jnp.round(u)
    z = f * f
    s = _SIN_C[5]
    for c in _SIN_C[4::-1]:
        s = s * z + c
    s = s * f
    c = _COS_C[6]
    for cc in _COS_C[5::-1]:
        c = c * z + cc
    onehot = (lab_ref[...] ==
              jax.lax.broadcasted_iota(jnp.int32, (br, 32), 1)
              ).astype(jnp.float32)
    emb = jax.lax.dot_general(
        onehot, tab_ref[...],
        dimension_numbers=(((1,), (0,)), ((), ())),
        precision=jax.lax.Precision.HIGHEST,
        preferred_element_type=jnp.float32)
    out_ref[...] = jnp.concatenate([s, c], axis=1) + emb


@functools.partial(jax.jit, static_argnames=("block_r",))
def _run(xy1, labels, w8, table, block_r=2048):
    R = xy1.shape[0]
    grid = (R // block_r,)
    return pl.pallas_call(
        _body,
        grid=grid,
        in_specs=[
            pl.BlockSpec((block_r, 8), lambda i: (i, 0)),
            pl.BlockSpec((block_r, 1), lambda i: (i, 0)),
            pl.BlockSpec((8, PHALF), lambda i: (0, 0)),
            pl.BlockSpec((32, PCH), lambda i: (0, 0)),
        ],
        out_specs=pl.BlockSpec((block_r, PCH), lambda i: (i, 0)),
        out_shape=jax.ShapeDtypeStruct((R, PCH), jnp.float32),
    )(xy1, labels, w8, table)


def kernel(branch_points, mid_points, branch_labels, mid_labels, pe_gauss,
           branch_table, mid_table):
    B = branch_points.shape[0]
    pts = jnp.concatenate([branch_points, mid_points], axis=1)
    R = B * NPTS
    xy = pts.reshape(R, 2)
    ones = jnp.ones((R, 1), jnp.float32)
    zeros = jnp.zeros((R, 5), jnp.float32)
    xy1 = jnp.concatenate([xy, ones, zeros], axis=1)
    labels = jnp.concatenate(
        [branch_labels, mid_labels + 16], axis=1).astype(jnp.int32)
    labels = labels.reshape(R, 1)
    g = pe_gauss.astype(jnp.float32)
    w8 = jnp.concatenate([
        g * jnp.float32(2.0 / IMG_SIZE),
        -(g[0:1] + g[1:2]),
        jnp.zeros((5, PHALF), jnp.float32),
    ], axis=0)
    table = jnp.concatenate([branch_table, mid_table], axis=0)
    block_r = next(b for b in (2048, 2000, 1000, 500, 200, 8) if R % b == 0)
    out = _run(xy1, labels, w8, table, block_r=block_r)
    return out.reshape(B, NPTS, PCH)


# speedup vs baseline: 1.1098x; 1.1098x over previous
import functools

import jax
import jax.numpy as jnp
import numpy as np
from jax.experimental import pallas as pl

IMG_SIZE = 1024.0
NB = 50
NM = 200
NCH = 5
PCH = 256
PHALF = 128

_SIN_C = tuple(np.float32(v) for v in (
    6.283183466376198, -41.34148035624613, 81.59765787614148,
    -76.59492821657112, 41.269929567669145, -12.372494818439662))
_COS_C = tuple(np.float32(v) for v in (
    0.999999992290297, -19.73920555404448, 64.93917223259542,
    -85.45116579292082, 60.176230338868066, -26.000527873748382,
    6.575611642718274))


def _body(bp_ref, mp_ref, bl_ref, ml_ref, g2_ref, c1_ref, bt_ref, mt_ref,
          out_ref):
    bb = bp_ref.shape[0]
    rows = bb * NB
    j = pl.program_id(1)
    is_branch = j == 0

    p = jnp.where(is_branch, bp_ref[...], mp_ref[...])
    u = jax.lax.dot_general(
        p.reshape(rows, 2), g2_ref[...],
        dimension_numbers=(((1,), (0,)), ((), ())),
        precision=jax.lax.Precision.HIGHEST,
        preferred_element_type=jnp.float32) + c1_ref[...]
    f = u - jnp.round(u)
    z = f * f
    s = _SIN_C[5]
    for c in _SIN_C[4::-1]:
        s = s * z + c
    s = s * f
    c = _COS_C[6]
    for cc in _COS_C[5::-1]:
        c = c * z + cc

    lab = jnp.where(is_branch, bl_ref[...], ml_ref[...])
    onehot = (lab.reshape(rows, 1) ==
              jax.lax.broadcasted_iota(jnp.int32, (rows, 16), 1)
              ).astype(jnp.float32)
    tab = jnp.where(is_branch, bt_ref[...], mt_ref[...])
    tab_hi = tab.astype(jnp.bfloat16).astype(jnp.float32)
    tab_lo = tab - tab_hi
    dn = (((1,), (0,)), ((), ()))
    emb = (jax.lax.dot_general(
        onehot, tab_hi, dimension_numbers=dn,
        preferred_element_type=jnp.float32) +
        jax.lax.dot_general(
        onehot, tab_lo, dimension_numbers=dn,
        preferred_element_type=jnp.float32))
    res = jnp.concatenate([s, c], axis=1) + emb
    out_ref[...] = res.reshape(bb, 1, NB, PCH)


@functools.partial(jax.jit, static_argnames=("block_b",))
def _run(bp4, mp4, bl3, ml3, g2, c1, btab, mtab, block_b=16):
    B = bp4.shape[0]
    grid = (B // block_b, NCH)
    out = pl.pallas_call(
        _body,
        grid=grid,
        in_specs=[
            pl.BlockSpec((block_b, 1, NB, 2), lambda i, j: (i, 0, 0, 0)),
            pl.BlockSpec((block_b, 1, NB, 2),
                         lambda i, j: (i, jnp.maximum(j - 1, 0), 0, 0)),
            pl.BlockSpec((block_b, 1, NB, 1), lambda i, j: (i, 0, 0, 0)),
            pl.BlockSpec((block_b, 1, NB, 1),
                         lambda i, j: (i, jnp.maximum(j - 1, 0), 0, 0)),
            pl.BlockSpec((2, PHALF), lambda i, j: (0, 0)),
            pl.BlockSpec((1, PHALF), lambda i, j: (0, 0)),
            pl.BlockSpec((16, PCH), lambda i, j: (0, 0)),
            pl.BlockSpec((16, PCH), lambda i, j: (0, 0)),
        ],
        out_specs=pl.BlockSpec((block_b, 1, NB, PCH),
                               lambda i, j: (i, j, 0, 0)),
        out_shape=jax.ShapeDtypeStruct((B, NCH, NB, PCH), jnp.float32),
    )(bp4, mp4, bl3, ml3, g2, c1, btab, mtab)
    return out.reshape(B, NCH * NB, PCH)


def kernel(branch_points, mid_points, branch_labels, mid_labels, pe_gauss,
           branch_table, mid_table):
    B = branch_points.shape[0]
    bp4 = branch_points.reshape(B, 1, NB, 2)
    mp4 = mid_points.reshape(B, NM // NB, NB, 2)
    bl3 = branch_labels.astype(jnp.int32).reshape(B, 1, NB, 1)
    ml3 = mid_labels.astype(jnp.int32).reshape(B, NM // NB, NB, 1)
    g = pe_gauss.astype(jnp.float32)
    g2 = g * jnp.float32(2.0 / IMG_SIZE)
    c1 = -(g[0:1] + g[1:2])
    block_b = 16 if B % 16 == 0 else 8
    return _run(bp4, mp4, bl3, ml3, g2, c1, branch_table, mid_table,
                block_b=block_b)
